# baseline (speedup 1.0000x reference)
import functools

import jax
import jax.numpy as jnp
from jax.experimental import pallas as pl
from jax.experimental.pallas import tpu as pltpu

_LAYERS = 3


def _diffusion_kernel(emb16_ref, g_ref, out_ref, g16_hbm,
                      wv, gv, buf_ref, acc_ref, ytmp_ref, wsem, rsem, *,
                      bm, nb):
    l = pl.program_id(0)
    i = pl.program_id(1)
    k = l * nb + i
    npairs = nb // 2
    tpairs = (_LAYERS - 1) * npairs
    half = jax.lax.rem(i, 2)
    p = jax.lax.div(i, 2)
    pg = (l - 1) * npairs + p
    ws = jax.lax.rem(i, 2)

    @pl.when(k == 0)
    def _init():
        buf_ref[0] = emb16_ref[...]

    @pl.when(jnp.logical_and(l == 0, i >= 2))
    def _wait_prev_write():
        pltpu.make_async_copy(
            wv.at[ws], g16_hbm.at[pl.ds((i - 2) * bm, bm), :],
            wsem.at[ws]).wait()

    @pl.when(l == 0)
    def _stage_and_write():
        wv[ws] = g_ref[...].astype(jnp.bfloat16)
        pltpu.make_async_copy(
            wv.at[ws], g16_hbm.at[pl.ds(i * bm, bm), :],
            wsem.at[ws]).start()

    @pl.when(jnp.logical_or(k == nb, k == nb + 1))
    def _wait_last_writes():
        pltpu.make_async_copy(
            wv.at[jax.lax.rem(k, 2)],
            g16_hbm.at[pl.ds((k - 2) * bm, bm), :],
            wsem.at[jax.lax.rem(k, 2)]).wait()

    @pl.when(k == nb - 1)
    def _bootstrap_reads():
        for j in range(2):
            pltpu.make_async_copy(
                g16_hbm.at[pl.ds(j * 2 * bm, 2 * bm), :], gv.at[j],
                rsem.at[j]).start()

    @pl.when(jnp.logical_and(
        jnp.logical_and(l >= 1, half == 0), pg + 2 < tpairs))
    def _prefetch_ahead():
        pn = jax.lax.rem(pg + 2, npairs)
        s = jax.lax.rem(pg + 2, 3)
        pltpu.make_async_copy(
            g16_hbm.at[pl.ds(pn * 2 * bm, 2 * bm), :], gv.at[s],
            rsem.at[s]).start()

    @pl.when(jnp.logical_and(l >= 1, half == 0))
    def _wait_read():
        s = jax.lax.rem(pg, 3)
        pltpu.make_async_copy(
            g16_hbm.at[pl.ds(p * 2 * bm, 2 * bm), :], gv.at[s],
            rsem.at[s]).wait()

    row = pl.ds(i * bm, bm)

    @pl.when(l == 0)
    def _compute0():
        y = jax.lax.dot_general(
            wv[ws], buf_ref[0], (((1,), (0,)), ((), ())),
            preferred_element_type=jnp.float32)
        buf_ref[1, row, :] = y.astype(jnp.bfloat16)
        new_acc = emb16_ref[row, :].astype(jnp.float32) + y
        acc_ref[row, :] = new_acc.astype(jnp.bfloat16)
        out_ref[...] = new_acc

    @pl.when(jnp.logical_and(l >= 1, half == 0))
    def _compute12():
        pair_rows = pl.ds(i * bm, 2 * bm)
        y = jax.lax.dot_general(
            gv[jax.lax.rem(pg, 3)],
            buf_ref[jax.lax.rem(l, 2)], (((1,), (0,)), ((), ())),
            preferred_element_type=jnp.float32)
        buf_ref[jax.lax.rem(l + 1, 2), pair_rows, :] = y.astype(jnp.bfloat16)
        new_acc = acc_ref[pair_rows, :].astype(jnp.float32) + y
        acc_ref[pair_rows, :] = new_acc.astype(jnp.bfloat16)
        out_ref[...] = new_acc[:bm]
        ytmp_ref[...] = new_acc[bm:]

    @pl.when(jnp.logical_and(l >= 1, half == 1))
    def _emit_second_half():
        out_ref[...] = ytmp_ref[...]


@jax.jit
def kernel(embedding, graph):
    n, d = embedding.shape
    bm = 200
    assert n % (2 * bm) == 0
    nb = n // bm

    return pl.pallas_call(
        functools.partial(_diffusion_kernel, bm=bm, nb=nb),
        grid=(_LAYERS, nb),
        in_specs=[
            pl.BlockSpec((n, d), lambda l, i: (0, 0)),
            pl.BlockSpec((bm, n), lambda l, i: (jnp.where(l == 0, i, 0), 0)),
        ],
        out_specs=[
            pl.BlockSpec((bm, d), lambda l, i: (i, 0)),
            pl.BlockSpec(memory_space=pltpu.MemorySpace.HBM),
        ],
        out_shape=[
            jax.ShapeDtypeStruct((n, d), jnp.float32),
            jax.ShapeDtypeStruct((n, n), jnp.bfloat16),
        ],
        scratch_shapes=[
            pltpu.VMEM((2, bm, n), jnp.bfloat16),
            pltpu.VMEM((3, 2 * bm, n), jnp.bfloat16),
            pltpu.VMEM((2, n, d), jnp.bfloat16),
            pltpu.VMEM((n, d), jnp.bfloat16),
            pltpu.VMEM((bm, d), jnp.float32),
            pltpu.SemaphoreType.DMA((2,)),
            pltpu.SemaphoreType.DMA((3,)),
        ],
    )(embedding.astype(jnp.bfloat16), graph)[0]


# speedup vs baseline: 1.0613x; 1.0613x over previous
import functools

import jax
import jax.numpy as jnp
from jax.experimental import pallas as pl
from jax.experimental.pallas import tpu as pltpu

_LAYERS = 3


def _diffusion_kernel(emb16_ref, g_ref, out_ref, g16_hbm,
                      gv, hv, buf_ref, acc_ref, p2_ref, yv_ref, wsem, rsem,
                      *, bm, nb, hsplit, bcols):
    l = pl.program_id(0)
    i = pl.program_id(1)
    k = l * nb + i
    n = nb * bm
    slot = jax.lax.rem(k, 4)

    @pl.when(k == 0)
    def _init():
        buf_ref[0] = emb16_ref[...]

    @pl.when(jnp.logical_and(l == 0, i >= 4))
    def _wait_prev_write():
        pltpu.make_async_copy(
            gv.at[slot], g16_hbm.at[pl.ds((i - 4) * bm, bm), :],
            wsem.at[slot]).wait()

    @pl.when(l == 0)
    def _stage_and_write():
        gv[slot] = g_ref[...].astype(jnp.bfloat16)
        pltpu.make_async_copy(
            gv.at[slot], g16_hbm.at[pl.ds(i * bm, bm), :],
            wsem.at[slot]).start()

    @pl.when(k == nb - 1)
    def _bootstrap_reads():
        for j in range(3):
            s = (nb + j) % 4
            pltpu.make_async_copy(
                gv.at[s], g16_hbm.at[pl.ds((nb - 4 + j) * bm, bm), :],
                wsem.at[s]).wait()
            pltpu.make_async_copy(
                g16_hbm.at[pl.ds(j * bm, bm), :], gv.at[s],
                rsem.at[s]).start()

    @pl.when(k == nb)
    def _wait_last_write():
        s = (nb - 1) % 4
        pltpu.make_async_copy(
            gv.at[s], g16_hbm.at[pl.ds((nb - 1) * bm, bm), :],
            wsem.at[s]).wait()

    m = k + 3
    ms = jax.lax.rem(m, 4)
    mblk = jax.lax.rem(m, nb)
    in_range = jnp.logical_and(k >= nb, k <= _LAYERS * nb - 4)
    m_half = jnp.logical_and(jax.lax.div(m, nb) == 1, mblk >= hsplit)

    @pl.when(jnp.logical_and(in_range, jnp.logical_not(m_half)))
    def _prefetch_full():
        pltpu.make_async_copy(
            g16_hbm.at[pl.ds(mblk * bm, bm), :], gv.at[ms],
            rsem.at[ms]).start()

    @pl.when(jnp.logical_and(in_range, m_half))
    def _prefetch_half():
        pltpu.make_async_copy(
            g16_hbm.at[pl.ds(mblk * bm, bm), pl.ds(bcols, n - bcols)],
            hv.at[ms], rsem.at[ms]).start()

    half_now = jnp.logical_and(l == 1, i >= hsplit)

    @pl.when(jnp.logical_and(l >= 1, jnp.logical_not(half_now)))
    def _wait_read_full():
        pltpu.make_async_copy(
            g16_hbm.at[pl.ds(i * bm, bm), :], gv.at[slot],
            rsem.at[slot]).wait()

    @pl.when(half_now)
    def _wait_read_half():
        pltpu.make_async_copy(
            g16_hbm.at[pl.ds(i * bm, bm), pl.ds(bcols, n - bcols)],
            hv.at[slot], rsem.at[slot]).wait()

    @pl.when(jnp.logical_not(half_now))
    def _dot_full():
        yv_ref[...] = jax.lax.dot_general(
            gv[slot], buf_ref[jax.lax.rem(l, 2)],
            (((1,), (0,)), ((), ())), preferred_element_type=jnp.float32)

    @pl.when(half_now)
    def _dot_half():
        yv_ref[...] = p2_ref[pl.ds((i - hsplit) * bm, bm), :].astype(
            jnp.float32) + jax.lax.dot_general(
                hv[slot], buf_ref[jax.lax.rem(l, 2), pl.ds(bcols, n - bcols), :],
                (((1,), (0,)), ((), ())), preferred_element_type=jnp.float32)

    @pl.when(jnp.logical_and(l == 0, i >= hsplit))
    def _partial_layer1():
        p2_ref[pl.ds((i - hsplit) * bm, bm), :] = jax.lax.dot_general(
            gv[slot, :, pl.ds(0, bcols)], buf_ref[1, pl.ds(0, bcols), :],
            (((1,), (0,)), ((), ())),
            preferred_element_type=jnp.float32).astype(jnp.bfloat16)

    y = yv_ref[...]
    row = pl.ds(i * bm, bm)
    buf_ref[jax.lax.rem(l + 1, 2), row, :] = y.astype(jnp.bfloat16)

    @pl.when(l == 0)
    def _acc_init():
        acc_ref[row, :] = emb16_ref[row, :].astype(jnp.float32) + y

    @pl.when(l > 0)
    def _acc_add():
        acc_ref[row, :] = acc_ref[row, :] + y

    out_ref[...] = acc_ref[row, :]


@jax.jit
def kernel(embedding, graph):
    n, d = embedding.shape
    bm = 200
    assert n % bm == 0
    nb = n // bm
    hsplit = nb // 2 + 1
    bcols = (hsplit * bm) // 128 * 128

    return pl.pallas_call(
        functools.partial(_diffusion_kernel, bm=bm, nb=nb, hsplit=hsplit,
                          bcols=bcols),
        grid=(_LAYERS, nb),
        in_specs=[
            pl.BlockSpec((n, d), lambda l, i: (0, 0)),
            pl.BlockSpec((bm, n), lambda l, i: (jnp.where(l == 0, i, 0), 0)),
        ],
        out_specs=[
            pl.BlockSpec((bm, d), lambda l, i: (i, 0)),
            pl.BlockSpec(memory_space=pltpu.MemorySpace.HBM),
        ],
        out_shape=[
            jax.ShapeDtypeStruct((n, d), jnp.float32),
            jax.ShapeDtypeStruct((n, n), jnp.bfloat16),
        ],
        scratch_shapes=[
            pltpu.VMEM((4, bm, n), jnp.bfloat16),
            pltpu.VMEM((4, bm, n - bcols), jnp.bfloat16),
            pltpu.VMEM((2, n, d), jnp.bfloat16),
            pltpu.VMEM((n, d), jnp.float32),
            pltpu.VMEM(((nb - hsplit) * bm, d), jnp.bfloat16),
            pltpu.VMEM((bm, d), jnp.float32),
            pltpu.SemaphoreType.DMA((4,)),
            pltpu.SemaphoreType.DMA((4,)),
        ],
    )(embedding.astype(jnp.bfloat16), graph)[0]


# speedup vs baseline: 1.0615x; 1.0003x over previous
import functools

import jax
import jax.numpy as jnp
from jax.experimental import pallas as pl
from jax.experimental.pallas import tpu as pltpu

_LAYERS = 3


def _diffusion_kernel(emb16_ref, g_ref, out_ref, g16_hbm,
                      gv, hv2, hv3, buf_ref, acc_ref, p2_ref, yv_ref,
                      wsem, rsem, *, bm, nb, tiers):
    l = pl.program_id(0)
    i = pl.program_id(1)
    k = l * nb + i
    n = nb * bm
    slot = jax.lax.rem(k, 4)
    h1 = tiers[0][0]
    narrow_refs = [hv2, hv3]
    bounds = [t[0] for t in tiers] + [nb]

    @pl.when(k == 0)
    def _init():
        buf_ref[0] = emb16_ref[...]

    @pl.when(jnp.logical_and(l == 0, i >= 4))
    def _wait_prev_write():
        pltpu.make_async_copy(
            gv.at[slot], g16_hbm.at[pl.ds((i - 4) * bm, bm), :],
            wsem.at[slot]).wait()

    @pl.when(l == 0)
    def _stage_and_write():
        gv[slot] = g_ref[...].astype(jnp.bfloat16)
        pltpu.make_async_copy(
            gv.at[slot], g16_hbm.at[pl.ds(i * bm, bm), :],
            wsem.at[slot]).start()

    @pl.when(k == nb - 1)
    def _bootstrap_reads():
        for j in range(3):
            s = (nb + j) % 4
            pltpu.make_async_copy(
                gv.at[s], g16_hbm.at[pl.ds((nb - 4 + j) * bm, bm), :],
                wsem.at[s]).wait()
            pltpu.make_async_copy(
                g16_hbm.at[pl.ds(j * bm, bm), :], gv.at[s],
                rsem.at[s]).start()

    @pl.when(k == nb)
    def _wait_last_write():
        s = (nb - 1) % 4
        pltpu.make_async_copy(
            gv.at[s], g16_hbm.at[pl.ds((nb - 1) * bm, bm), :],
            wsem.at[s]).wait()

    m = k + 3
    ms = jax.lax.rem(m, 4)
    mblk = jax.lax.rem(m, nb)
    in_range = jnp.logical_and(k >= nb, k <= _LAYERS * nb - 4)
    m_l1 = jax.lax.div(m, nb) == 1

    @pl.when(jnp.logical_and(
        in_range, jnp.logical_not(jnp.logical_and(m_l1, mblk >= h1))))
    def _prefetch_full():
        pltpu.make_async_copy(
            g16_hbm.at[pl.ds(mblk * bm, bm), :], gv.at[ms],
            rsem.at[ms]).start()

    for t, ((lo, bcol), hi) in enumerate(zip(tiers, bounds[1:])):
        @pl.when(jnp.logical_and(
            jnp.logical_and(in_range, m_l1),
            jnp.logical_and(mblk >= lo, mblk < hi)))
        def _prefetch_narrow(bcol=bcol, href=narrow_refs[t]):
            pltpu.make_async_copy(
                g16_hbm.at[pl.ds(mblk * bm, bm), pl.ds(bcol, n - bcol)],
                href.at[ms], rsem.at[ms]).start()

    narrow_now = jnp.logical_and(l == 1, i >= h1)

    @pl.when(jnp.logical_and(l >= 1, jnp.logical_not(narrow_now)))
    def _wait_read_full():
        pltpu.make_async_copy(
            g16_hbm.at[pl.ds(i * bm, bm), :], gv.at[slot],
            rsem.at[slot]).wait()

    @pl.when(jnp.logical_not(narrow_now))
    def _dot_full():
        yv_ref[...] = jax.lax.dot_general(
            gv[slot], buf_ref[jax.lax.rem(l, 2)],
            (((1,), (0,)), ((), ())), preferred_element_type=jnp.float32)

    for t, ((lo, bcol), hi) in enumerate(zip(tiers, bounds[1:])):
        in_tier = jnp.logical_and(i >= lo, i < hi)

        @pl.when(jnp.logical_and(narrow_now, in_tier))
        def _wait_and_dot_narrow(bcol=bcol, href=narrow_refs[t]):
            pltpu.make_async_copy(
                g16_hbm.at[pl.ds(i * bm, bm), pl.ds(bcol, n - bcol)],
                href.at[slot], rsem.at[slot]).wait()
            yv_ref[...] = p2_ref[pl.ds((i - h1) * bm, bm), :].astype(
                jnp.float32) + jax.lax.dot_general(
                    href[slot],
                    buf_ref[jax.lax.rem(l, 2), pl.ds(bcol, n - bcol), :],
                    (((1,), (0,)), ((), ())),
                    preferred_element_type=jnp.float32)

        @pl.when(jnp.logical_and(l == 0, in_tier))
        def _partial_layer1(bcol=bcol):
            p2_ref[pl.ds((i - h1) * bm, bm), :] = jax.lax.dot_general(
                gv[slot, :, pl.ds(0, bcol)], buf_ref[1, pl.ds(0, bcol), :],
                (((1,), (0,)), ((), ())),
                preferred_element_type=jnp.float32).astype(jnp.bfloat16)

    y = yv_ref[...]
    row = pl.ds(i * bm, bm)
    buf_ref[jax.lax.rem(l + 1, 2), row, :] = y.astype(jnp.bfloat16)

    @pl.when(l == 0)
    def _acc_init():
        acc_ref[row, :] = emb16_ref[row, :].astype(jnp.float32) + y

    @pl.when(l > 0)
    def _acc_add():
        acc_ref[row, :] = acc_ref[row, :] + y

    out_ref[...] = acc_ref[row, :]


@jax.jit
def kernel(embedding, graph):
    n, d = embedding.shape
    bm = 200
    assert n % bm == 0
    nb = n // bm
    los = [nb // 2 + 1, 3 * nb // 4 + 1]
    tiers = tuple((lo, (lo * bm) // 128 * 128) for lo in los)
    h1 = tiers[0][0]

    return pl.pallas_call(
        functools.partial(_diffusion_kernel, bm=bm, nb=nb, tiers=tiers),
        grid=(_LAYERS, nb),
        in_specs=[
            pl.BlockSpec((n, d), lambda l, i: (0, 0)),
            pl.BlockSpec((bm, n), lambda l, i: (jnp.where(l == 0, i, 0), 0)),
        ],
        out_specs=[
            pl.BlockSpec((bm, d), lambda l, i: (i, 0)),
            pl.BlockSpec(memory_space=pltpu.MemorySpace.HBM),
        ],
        out_shape=[
            jax.ShapeDtypeStruct((n, d), jnp.float32),
            jax.ShapeDtypeStruct((n, n), jnp.bfloat16),
        ],
        scratch_shapes=[
            pltpu.VMEM((4, bm, n), jnp.bfloat16),
            pltpu.VMEM((4, bm, n - tiers[0][1]), jnp.bfloat16),
            pltpu.VMEM((4, bm, n - tiers[1][1]), jnp.bfloat16),
            pltpu.VMEM((2, n, d), jnp.bfloat16),
            pltpu.VMEM((n, d), jnp.float32),
            pltpu.VMEM(((nb - h1) * bm, d), jnp.bfloat16),
            pltpu.VMEM((bm, d), jnp.float32),
            pltpu.SemaphoreType.DMA((4,)),
            pltpu.SemaphoreType.DMA((4,)),
        ],
    )(embedding.astype(jnp.bfloat16), graph)[0]


# speedup vs baseline: 1.0662x; 1.0044x over previous
import functools

import jax
import jax.numpy as jnp
from jax.experimental import pallas as pl
from jax.experimental.pallas import tpu as pltpu

_LAYERS = 3


def _diffusion_kernel(emb16_ref, g_ref, out_ref, g16_hbm,
                      gv, hv2, hv3, buf_ref, acc_ref, p2_ref, yv_ref,
                      wsem, rsem, *, bm, nb, tiers):
    l = pl.program_id(0)
    i = pl.program_id(1)
    k = l * nb + i
    n = nb * bm
    slot = jax.lax.rem(k, 4)
    h1 = tiers[0][0]
    narrow_refs = [hv2, hv3]
    bounds = [t[0] for t in tiers] + [nb]

    @pl.when(k == 0)
    def _init():
        buf_ref[0] = emb16_ref[...]

    @pl.when(jnp.logical_and(l == 0, i >= 4))
    def _wait_prev_write():
        pltpu.make_async_copy(
            gv.at[slot], g16_hbm.at[pl.ds((i - 4) * bm, bm), :],
            wsem.at[slot]).wait()

    @pl.when(l == 0)
    def _stage_and_write():
        gv[slot] = g_ref[...].astype(jnp.bfloat16)
        pltpu.make_async_copy(
            gv.at[slot], g16_hbm.at[pl.ds(i * bm, bm), :],
            wsem.at[slot]).start()

    @pl.when(k == nb - 1)
    def _bootstrap_reads():
        for j in range(3):
            s = (nb + j) % 4
            pltpu.make_async_copy(
                gv.at[s], g16_hbm.at[pl.ds((nb - 4 + j) * bm, bm), :],
                wsem.at[s]).wait()
            pltpu.make_async_copy(
                g16_hbm.at[pl.ds(j * bm, bm), :], gv.at[s],
                rsem.at[s]).start()

    @pl.when(k == nb)
    def _wait_last_write():
        s = (nb - 1) % 4
        pltpu.make_async_copy(
            gv.at[s], g16_hbm.at[pl.ds((nb - 1) * bm, bm), :],
            wsem.at[s]).wait()

    m = k + 3
    ms = jax.lax.rem(m, 4)
    mblk = jax.lax.rem(m, nb)
    in_range = jnp.logical_and(k >= nb, k <= _LAYERS * nb - 4)
    m_l1 = jax.lax.div(m, nb) == 1

    @pl.when(jnp.logical_and(
        in_range, jnp.logical_not(jnp.logical_and(m_l1, mblk >= h1))))
    def _prefetch_full():
        pltpu.make_async_copy(
            g16_hbm.at[pl.ds(mblk * bm, bm), :], gv.at[ms],
            rsem.at[ms]).start()

    for t, ((lo, bcol), hi) in enumerate(zip(tiers, bounds[1:])):
        @pl.when(jnp.logical_and(
            jnp.logical_and(in_range, m_l1),
            jnp.logical_and(mblk >= lo, mblk < hi)))
        def _prefetch_narrow(bcol=bcol, href=narrow_refs[t]):
            pltpu.make_async_copy(
                g16_hbm.at[pl.ds(mblk * bm, bm), pl.ds(bcol, n - bcol)],
                href.at[ms], rsem.at[ms]).start()

    narrow_now = jnp.logical_and(l == 1, i >= h1)

    @pl.when(jnp.logical_and(l >= 1, jnp.logical_not(narrow_now)))
    def _wait_read_full():
        pltpu.make_async_copy(
            g16_hbm.at[pl.ds(i * bm, bm), :], gv.at[slot],
            rsem.at[slot]).wait()

    @pl.when(jnp.logical_not(narrow_now))
    def _dot_full():
        yv_ref[...] = jax.lax.dot_general(
            gv[slot], buf_ref[jax.lax.rem(l, 2)],
            (((1,), (0,)), ((), ())), preferred_element_type=jnp.float32)

    for t, ((lo, bcol), hi) in enumerate(zip(tiers, bounds[1:])):
        in_tier = jnp.logical_and(i >= lo, i < hi)

        @pl.when(jnp.logical_and(narrow_now, in_tier))
        def _wait_and_dot_narrow(bcol=bcol, href=narrow_refs[t]):
            pltpu.make_async_copy(
                g16_hbm.at[pl.ds(i * bm, bm), pl.ds(bcol, n - bcol)],
                href.at[slot], rsem.at[slot]).wait()
            yv_ref[...] = p2_ref[pl.ds((i - h1) * bm, bm), :].astype(
                jnp.float32) + jax.lax.dot_general(
                    href[slot],
                    buf_ref[jax.lax.rem(l, 2), pl.ds(bcol, n - bcol), :],
                    (((1,), (0,)), ((), ())),
                    preferred_element_type=jnp.float32)

        @pl.when(jnp.logical_and(l == 0, in_tier))
        def _partial_layer1(bcol=bcol):
            p2_ref[pl.ds((i - h1) * bm, bm), :] = jax.lax.dot_general(
                gv[slot, :, pl.ds(0, bcol)], buf_ref[1, pl.ds(0, bcol), :],
                (((1,), (0,)), ((), ())),
                preferred_element_type=jnp.float32).astype(jnp.bfloat16)

    y = yv_ref[...]
    row = pl.ds(i * bm, bm)
    buf_ref[jax.lax.rem(l + 1, 2), row, :] = y.astype(jnp.bfloat16)

    @pl.when(l == 0)
    def _acc_init():
        acc_ref[row, :] = emb16_ref[row, :].astype(jnp.float32) + y

    @pl.when(l > 0)
    def _acc_add():
        acc_ref[row, :] = acc_ref[row, :] + y

    out_ref[...] = acc_ref[row, :]


@jax.jit
def kernel(embedding, graph):
    n, d = embedding.shape
    bm = 200
    assert n % bm == 0
    nb = n // bm
    los = [nb // 2 + 1, 3 * nb // 4 + 1]
    tiers = tuple((lo, (lo * bm) // 128 * 128) for lo in los)
    h1 = tiers[0][0]

    return pl.pallas_call(
        functools.partial(_diffusion_kernel, bm=bm, nb=nb, tiers=tiers),
        grid=(_LAYERS, nb),
        in_specs=[
            pl.BlockSpec((n, d), lambda l, i: (0, 0)),
            pl.BlockSpec((bm, n), lambda l, i: (jnp.where(l == 0, i, 0), 0)),
        ],
        out_specs=[
            pl.BlockSpec((bm, d),
                         lambda l, i: (jnp.where(l == _LAYERS - 1, i, 0), 0)),
            pl.BlockSpec(memory_space=pltpu.MemorySpace.HBM),
        ],
        out_shape=[
            jax.ShapeDtypeStruct((n, d), jnp.float32),
            jax.ShapeDtypeStruct((n, n), jnp.bfloat16),
        ],
        scratch_shapes=[
            pltpu.VMEM((4, bm, n), jnp.bfloat16),
            pltpu.VMEM((4, bm, n - tiers[0][1]), jnp.bfloat16),
            pltpu.VMEM((4, bm, n - tiers[1][1]), jnp.bfloat16),
            pltpu.VMEM((2, n, d), jnp.bfloat16),
            pltpu.VMEM((n, d), jnp.float32),
            pltpu.VMEM(((nb - h1) * bm, d), jnp.bfloat16),
            pltpu.VMEM((bm, d), jnp.float32),
            pltpu.SemaphoreType.DMA((4,)),
            pltpu.SemaphoreType.DMA((4,)),
        ],
    )(embedding.astype(jnp.bfloat16), graph)[0]
